# Initial kernel scaffold; baseline (speedup 1.0000x reference)
import jax
import jax.numpy as jnp
from jax.experimental import pallas as pl


def kernel(node_attrs, batch, ptr, layer_output_energies, atomic_energies):
    raise NotImplementedError("write your pallas kernel here")


# speedup vs baseline: 1.3579x; 1.3579x over previous
import functools

import jax
import jax.numpy as jnp
from jax import lax
from jax.experimental import pallas as pl
from jax.experimental.pallas import tpu as pltpu
from jax.experimental.pallas import tpu_sc as plsc

N_LANES = 16
NUM_CORES = 2
NUM_SUBCORES = 16
NUM_WORKERS = NUM_CORES * NUM_SUBCORES


@functools.lru_cache(maxsize=None)
def _build_sc_kernel(n_nodes: int, n_elements: int, num_graphs: int):
    chunk = -(-n_nodes // NUM_WORKERS)
    chunk = -(-chunk // 8) * 8
    iters = -(-chunk // N_LANES)
    buf_rows = iters * N_LANES
    g_rows = num_graphs // 32

    mesh = plsc.VectorSubcoreMesh(core_axis_name="c", subcore_axis_name="s")

    @functools.partial(
        pl.kernel,
        mesh=mesh,
        compiler_params=pltpu.CompilerParams(needs_layout_passes=False),
        out_type=[
            jax.ShapeDtypeStruct((NUM_CORES, num_graphs), jnp.float32),
            jax.ShapeDtypeStruct((n_nodes,), jnp.float32),
        ],
        scratch_types=[
            pltpu.VMEM((buf_rows * n_elements,), jnp.float32),
            pltpu.VMEM((buf_rows,), jnp.int32),
            pltpu.VMEM((buf_rows,), jnp.float32),
            pltpu.VMEM((buf_rows,), jnp.float32),
            pltpu.VMEM((N_LANES,), jnp.float32),
            pltpu.VMEM((g_rows, 32), jnp.float32),
            pltpu.VMEM((NUM_SUBCORES, 32), jnp.float32),
            pltpu.VMEM((32,), jnp.float32),
            pltpu.VMEM_SHARED((NUM_SUBCORES, g_rows, 32), jnp.float32),
        ],
    )
    def sc_kernel(attrs_hbm, batch_hbm, loe_hbm, ae_hbm,
                  partial_hbm, ne_hbm,
                  attrs_v, batch_v, loe_v, ne_v, ae_v, seg_v, comb_v, out_v,
                  shared):
        c = lax.axis_index("c")
        s = lax.axis_index("s")
        w = s * NUM_CORES + c
        base = jnp.minimum(chunk * w, n_nodes - chunk)
        lo = chunk * w - base

        pltpu.sync_copy(attrs_hbm.at[pl.ds(base * n_elements, chunk * n_elements)],
                        attrs_v.at[pl.ds(0, chunk * n_elements)])
        pltpu.sync_copy(batch_hbm.at[pl.ds(base, chunk)],
                        batch_v.at[pl.ds(0, chunk)])
        pltpu.sync_copy(loe_hbm.at[pl.ds(base, chunk)],
                        loe_v.at[pl.ds(0, chunk)])
        pltpu.sync_copy(ae_hbm, ae_v.at[pl.ds(0, n_elements)])

        zero16 = jnp.zeros((N_LANES,), jnp.float32)
        for r in range(g_rows):
            for h in range(32 // N_LANES):
                seg_v[r, pl.ds(h * N_LANES, N_LANES)] = zero16

        ae_vec = ae_v[pl.ds(0, N_LANES)]
        ae_scalars = [ae_vec[e] for e in range(n_elements)]
        iota = lax.iota(jnp.int32, N_LANES)

        def step(i, carry):
            pos = i * N_LANES + iota
            acc = loe_v[pl.ds(i * N_LANES, N_LANES)]
            flat_base = pos * n_elements
            for e in range(n_elements):
                vals = plsc.load_gather(attrs_v, [flat_base + e])
                acc = acc + vals * ae_scalars[e]
            ne_v[pl.ds(i * N_LANES, N_LANES)] = acc
            b = batch_v[pl.ds(i * N_LANES, N_LANES)]
            own = (pos >= lo) & (pos < chunk)
            plsc.addupdate_scatter(
                seg_v, [b >> 5, b & 31], acc, mask=own)
            return carry

        lax.fori_loop(0, iters, step, 0)

        pltpu.sync_copy(ne_v.at[pl.ds(0, chunk)],
                        ne_hbm.at[pl.ds(base, chunk)])

        pltpu.sync_copy(seg_v, shared.at[s])
        plsc.subcore_barrier()
        pltpu.sync_copy(shared.at[:, s, :], comb_v)
        for h in range(32 // N_LANES):
            accv = zero16
            for r in range(NUM_SUBCORES):
                accv = accv + comb_v[r, pl.ds(h * N_LANES, N_LANES)]
            out_v[pl.ds(h * N_LANES, N_LANES)] = accv
        pltpu.sync_copy(out_v, partial_hbm.at[c, pl.ds(s * 32, 32)])

    return sc_kernel


def kernel(node_attrs, batch, ptr, layer_output_energies, atomic_energies):
    n_nodes, n_elements = node_attrs.shape
    num_graphs = ptr.shape[0] - 1
    sc = _build_sc_kernel(n_nodes, n_elements, num_graphs)
    attrs_flat = node_attrs.reshape(-1)
    partials, node_energy = sc(attrs_flat, batch, layer_output_energies,
                               atomic_energies)
    total_energy = partials[0] + partials[1]
    return total_energy, node_energy
